# baseline (speedup 1.0000x reference)
import functools

import jax
import jax.numpy as jnp
from jax import lax
from jax.experimental import pallas as pl
from jax.experimental.pallas import tpu as pltpu

_B, _D, _K = 16384, 64, 1024
_BETA = 0.25
_TB = 2048
_GRID = _B // _TB


def _vq_body(x_ref, w_ref, idx_ref, q_ref, loss_ref, w2_ref, wb_ref):
    i = pl.program_id(0)
    x = x_ref[...]

    @pl.when(i == 0)
    def _():
        w = w_ref[...]
        w2_ref[...] = jnp.sum(w * w, axis=1, keepdims=True).T
        wb_ref[...] = w.astype(jnp.bfloat16)

    x2 = jnp.sum(x * x, axis=1, keepdims=True)
    mm = jnp.matmul(x, w_ref[...].T)
    d = x2 + w2_ref[...] - 2.0 * mm

    m = jnp.min(d, axis=1, keepdims=True)
    ids = lax.broadcasted_iota(jnp.int32, d.shape, 1)
    idx = jnp.min(jnp.where(d == m, ids, _K), axis=1)
    idx_ref[...] = idx[:, None]

    one_hot = (ids == idx[:, None]).astype(jnp.bfloat16)
    q = jnp.dot(one_hot, wb_ref[...], preferred_element_type=jnp.float32)
    q_ref[...] = x + (q - x)

    part = jnp.sum((q - x) * (q - x)).reshape(1, 1)

    @pl.when(i == 0)
    def _():
        loss_ref[...] = jnp.zeros((1, 1), jnp.float32)

    loss_ref[...] += part

    @pl.when(i == _GRID - 1)
    def _():
        s = loss_ref[...] / jnp.float32(_B * _D)
        loss_ref[...] = s * _BETA + s


@jax.jit
def kernel(x, W):
    idx, q, loss = pl.pallas_call(
        _vq_body,
        grid=(_GRID,),
        in_specs=[
            pl.BlockSpec((_TB, _D), lambda i: (i, 0)),
            pl.BlockSpec((_K, _D), lambda i: (0, 0)),
        ],
        out_specs=[
            pl.BlockSpec((_TB, 1), lambda i: (i, 0)),
            pl.BlockSpec((_TB, _D), lambda i: (i, 0)),
            pl.BlockSpec((1, 1), lambda i: (0, 0)),
        ],
        out_shape=[
            jax.ShapeDtypeStruct((_B, 1), jnp.int32),
            jax.ShapeDtypeStruct((_B, _D), jnp.float32),
            jax.ShapeDtypeStruct((1, 1), jnp.float32),
        ],
        scratch_shapes=[
            pltpu.VMEM((1, _K), jnp.float32),
            pltpu.VMEM((_K, _D), jnp.bfloat16),
        ],
    )(x, W)
    return idx, q, loss[0, 0]


# speedup vs baseline: 2.0562x; 2.0562x over previous
import functools

import jax
import jax.numpy as jnp
from jax import lax
from jax.experimental import pallas as pl
from jax.experimental.pallas import tpu as pltpu

_B, _D, _K = 16384, 64, 1024
_BETA = 0.25
_TB = 2048
_GRID = _B // _TB


def _vq_body(xt_ref, wt_ref, idx_ref, qt_ref, loss_ref, w2_ref, wb_ref):
    i = pl.program_id(0)
    xt = xt_ref[...]

    @pl.when(i == 0)
    def _():
        wt = wt_ref[...]
        w2_ref[...] = jnp.sum(wt * wt, axis=0, keepdims=True).T
        wb_ref[...] = wt.astype(jnp.bfloat16)

    x2 = jnp.sum(xt * xt, axis=0, keepdims=True)
    mm = lax.dot_general(wt_ref[...], xt,
                         (((0,), (0,)), ((), ())))
    d = x2 + w2_ref[...] - 2.0 * mm

    m = jnp.min(d, axis=0, keepdims=True)
    ids = lax.broadcasted_iota(jnp.int32, d.shape, 0)
    idx = jnp.min(jnp.where(d == m, ids, _K), axis=0)
    idx_ref[...] = idx[None, :]

    one_hot = (ids == idx[None, :]).astype(jnp.bfloat16)
    q = lax.dot_general(wb_ref[...], one_hot,
                        (((1,), (0,)), ((), ())),
                        preferred_element_type=jnp.float32)
    qt_ref[...] = xt + (q - xt)

    part = jnp.sum((q - xt) * (q - xt)).reshape(1, 1)

    @pl.when(i == 0)
    def _():
        loss_ref[...] = jnp.zeros((1, 1), jnp.float32)

    loss_ref[...] += part

    @pl.when(i == _GRID - 1)
    def _():
        s = loss_ref[...] / jnp.float32(_B * _D)
        loss_ref[...] = s * _BETA + s


@jax.jit
def kernel(x, W):
    idxt, qt, loss = pl.pallas_call(
        _vq_body,
        grid=(_GRID,),
        in_specs=[
            pl.BlockSpec((_D, _TB), lambda i: (0, i)),
            pl.BlockSpec((_D, _K), lambda i: (0, 0)),
        ],
        out_specs=[
            pl.BlockSpec((1, _TB), lambda i: (0, i)),
            pl.BlockSpec((_D, _TB), lambda i: (0, i)),
            pl.BlockSpec((1, 1), lambda i: (0, 0)),
        ],
        out_shape=[
            jax.ShapeDtypeStruct((1, _B), jnp.int32),
            jax.ShapeDtypeStruct((_D, _B), jnp.float32),
            jax.ShapeDtypeStruct((1, 1), jnp.float32),
        ],
        scratch_shapes=[
            pltpu.VMEM((_K, 1), jnp.float32),
            pltpu.VMEM((_D, _K), jnp.bfloat16),
        ],
    )(x.T, W.T)
    return idxt.T, qt.T, loss[0, 0]


# speedup vs baseline: 2.1340x; 1.0378x over previous
import functools

import jax
import jax.numpy as jnp
from jax import lax
from jax.experimental import pallas as pl
from jax.experimental.pallas import tpu as pltpu

_B, _D, _K = 16384, 64, 1024
_BETA = 0.25
_TB = 4096
_GRID = _B // _TB


def _vq_body(xt_ref, wt_ref, idx_ref, qt_ref, loss_ref, w2_ref, wb_ref):
    i = pl.program_id(0)
    xt = xt_ref[...]

    @pl.when(i == 0)
    def _():
        wt = wt_ref[...]
        w2_ref[...] = jnp.sum(wt * wt, axis=0, keepdims=True).T
        wb_ref[...] = wt.astype(jnp.bfloat16)

    x2 = jnp.sum(xt * xt, axis=0, keepdims=True)
    mm = lax.dot_general(wt_ref[...], xt,
                         (((0,), (0,)), ((), ())))
    d = x2 + w2_ref[...] - 2.0 * mm

    m = jnp.min(d, axis=0, keepdims=True)
    ids = lax.broadcasted_iota(jnp.int32, d.shape, 0)
    idx = jnp.min(jnp.where(d == m, ids, _K), axis=0)
    idx_ref[...] = idx[None, :]

    one_hot = (ids == idx[None, :]).astype(jnp.bfloat16)
    q = lax.dot_general(wb_ref[...], one_hot,
                        (((1,), (0,)), ((), ())),
                        preferred_element_type=jnp.float32)
    qt_ref[...] = xt + (q - xt)

    part = jnp.sum((q - xt) * (q - xt)).reshape(1, 1)

    @pl.when(i == 0)
    def _():
        loss_ref[...] = jnp.zeros((1, 1), jnp.float32)

    loss_ref[...] += part

    @pl.when(i == _GRID - 1)
    def _():
        s = loss_ref[...] / jnp.float32(_B * _D)
        loss_ref[...] = s * _BETA + s


@jax.jit
def kernel(x, W):
    idxt, qt, loss = pl.pallas_call(
        _vq_body,
        grid=(_GRID,),
        in_specs=[
            pl.BlockSpec((_D, _TB), lambda i: (0, i)),
            pl.BlockSpec((_D, _K), lambda i: (0, 0)),
        ],
        out_specs=[
            pl.BlockSpec((1, _TB), lambda i: (0, i)),
            pl.BlockSpec((_D, _TB), lambda i: (0, i)),
            pl.BlockSpec((1, 1), lambda i: (0, 0)),
        ],
        out_shape=[
            jax.ShapeDtypeStruct((1, _B), jnp.int32),
            jax.ShapeDtypeStruct((_D, _B), jnp.float32),
            jax.ShapeDtypeStruct((1, 1), jnp.float32),
        ],
        scratch_shapes=[
            pltpu.VMEM((_K, 1), jnp.float32),
            pltpu.VMEM((_D, _K), jnp.bfloat16),
        ],
    )(x.T, W.T)
    return idxt.T, qt.T, loss[0, 0]


# speedup vs baseline: 2.1689x; 1.0163x over previous
import functools

import jax
import jax.numpy as jnp
from jax import lax
from jax.experimental import pallas as pl
from jax.experimental.pallas import tpu as pltpu

_B, _D, _K = 16384, 64, 1024
_BETA = 0.25
_TB = 8192
_GRID = _B // _TB


def _vq_body(xt_ref, wt_ref, idx_ref, qt_ref, loss_ref, w2_ref, wb_ref):
    i = pl.program_id(0)
    xt = xt_ref[...]

    @pl.when(i == 0)
    def _():
        wt = wt_ref[...]
        w2_ref[...] = jnp.sum(wt * wt, axis=0, keepdims=True).T
        wb_ref[...] = wt.astype(jnp.bfloat16)

    x2 = jnp.sum(xt * xt, axis=0, keepdims=True)
    mm = lax.dot_general(wt_ref[...], xt,
                         (((0,), (0,)), ((), ())))
    d = x2 + w2_ref[...] - 2.0 * mm

    m = jnp.min(d, axis=0, keepdims=True)
    ids = lax.broadcasted_iota(jnp.int32, d.shape, 0)
    idx = jnp.min(jnp.where(d == m, ids, _K), axis=0)
    idx_ref[...] = idx[None, :]

    one_hot = (ids == idx[None, :]).astype(jnp.bfloat16)
    q = lax.dot_general(wb_ref[...], one_hot,
                        (((1,), (0,)), ((), ())),
                        preferred_element_type=jnp.float32)
    qt_ref[...] = xt + (q - xt)

    part = jnp.sum((q - xt) * (q - xt)).reshape(1, 1)

    @pl.when(i == 0)
    def _():
        loss_ref[...] = jnp.zeros((1, 1), jnp.float32)

    loss_ref[...] += part

    @pl.when(i == _GRID - 1)
    def _():
        s = loss_ref[...] / jnp.float32(_B * _D)
        loss_ref[...] = s * _BETA + s


@jax.jit
def kernel(x, W):
    idxt, qt, loss = pl.pallas_call(
        _vq_body,
        grid=(_GRID,),
        in_specs=[
            pl.BlockSpec((_D, _TB), lambda i: (0, i)),
            pl.BlockSpec((_D, _K), lambda i: (0, 0)),
        ],
        out_specs=[
            pl.BlockSpec((1, _TB), lambda i: (0, i)),
            pl.BlockSpec((_D, _TB), lambda i: (0, i)),
            pl.BlockSpec((1, 1), lambda i: (0, 0)),
        ],
        out_shape=[
            jax.ShapeDtypeStruct((1, _B), jnp.int32),
            jax.ShapeDtypeStruct((_D, _B), jnp.float32),
            jax.ShapeDtypeStruct((1, 1), jnp.float32),
        ],
        scratch_shapes=[
            pltpu.VMEM((_K, 1), jnp.float32),
            pltpu.VMEM((_D, _K), jnp.bfloat16),
        ],
    )(x.T, W.T)
    return idxt.T, qt.T, loss[0, 0]


# speedup vs baseline: 2.1741x; 1.0024x over previous
import jax
import jax.numpy as jnp
from jax import lax
from jax.experimental import pallas as pl
from jax.experimental.pallas import tpu as pltpu

_B, _D, _K = 16384, 64, 1024
_BETA = 0.25
_TB = 8192
_GRID = _B // _TB


def _vq_body(xt_ref, wt_ref, idx_ref, qt_ref, loss_ref, w2_ref, wb_ref):
    i = pl.program_id(0)
    xt = xt_ref[...]

    @pl.when(i == 0)
    def _():
        wt = wt_ref[...]
        w2_ref[...] = jnp.sum(wt * wt, axis=0, keepdims=True).T
        wb_ref[...] = wt.astype(jnp.bfloat16)

    x2 = jnp.sum(xt * xt, axis=0, keepdims=True)
    mm = lax.dot_general(wt_ref[...], xt,
                         (((0,), (0,)), ((), ())))
    d = x2 + w2_ref[...] - 2.0 * mm

    m = jnp.min(d, axis=0, keepdims=True)
    ids = lax.broadcasted_iota(jnp.int32, d.shape, 0)
    idx = jnp.min(jnp.where(d == m, ids, _K), axis=0)
    idx_ref[...] = idx[None, :]

    one_hot = (ids == idx[None, :]).astype(jnp.bfloat16)
    q = lax.dot_general(wb_ref[...], one_hot,
                        (((1,), (0,)), ((), ())),
                        preferred_element_type=jnp.float32)
    qt_ref[...] = xt + (q - xt)

    part = jnp.sum((q - xt) * (q - xt)).reshape(1, 1)

    @pl.when(i == 0)
    def _():
        loss_ref[...] = jnp.zeros((1, 1), jnp.float32)

    loss_ref[...] += part

    @pl.when(i == _GRID - 1)
    def _():
        s = loss_ref[...] / jnp.float32(_B * _D)
        loss_ref[...] = s * _BETA + s


@jax.jit
def kernel(x, W):
    idxt, qt, loss = pl.pallas_call(
        _vq_body,
        grid=(_GRID,),
        in_specs=[
            pl.BlockSpec((_D, _TB), lambda i: (0, i)),
            pl.BlockSpec((_D, _K), lambda i: (0, 0)),
        ],
        out_specs=[
            pl.BlockSpec((1, _TB), lambda i: (0, i)),
            pl.BlockSpec((_D, _TB), lambda i: (0, i)),
            pl.BlockSpec((1, 1), lambda i: (0, 0)),
        ],
        out_shape=[
            jax.ShapeDtypeStruct((1, _B), jnp.int32),
            jax.ShapeDtypeStruct((_D, _B), jnp.float32),
            jax.ShapeDtypeStruct((1, 1), jnp.float32),
        ],
        scratch_shapes=[
            pltpu.VMEM((_K, 1), jnp.float32),
            pltpu.VMEM((_D, _K), jnp.bfloat16),
        ],
    )(x.T, W.T)
    return idxt.T, qt.T, loss[0, 0]


# speedup vs baseline: 2.4673x; 1.1348x over previous
import jax
import jax.numpy as jnp
from jax import lax
from jax.experimental import pallas as pl
from jax.experimental.pallas import tpu as pltpu

_B, _D, _K = 16384, 64, 1024
_BETA = 0.25
_TB = 8192
_GRID = _B // _TB


def _vq_body(xt_ref, wt_ref, idx_ref, qt_ref, loss_ref, w2_ref, wb_ref):
    i = pl.program_id(0)
    xt = xt_ref[...]

    @pl.when(i == 0)
    def _():
        wt = wt_ref[...]
        w2_ref[...] = jnp.sum(wt * wt, axis=0, keepdims=True).T
        wb_ref[...] = wt.astype(jnp.bfloat16)

    x2 = jnp.sum(xt * xt, axis=0, keepdims=True)
    mm = lax.dot_general(wt_ref[...], xt,
                         (((0,), (0,)), ((), ())))
    w2 = w2_ref[...]

    s = _K // 2
    lo = x2 + w2[:s] - 2.0 * mm[:s]
    hi = x2 + w2[s:] - 2.0 * mm[s:]
    mask = hi < lo
    val = jnp.minimum(lo, hi)
    acc = jnp.where(mask, jnp.int32(s), jnp.int32(0))
    s //= 2
    while s >= 8:
        mask = val[s:] < val[:s]
        acc = jnp.where(mask, acc[s:] + jnp.int32(s), acc[:s])
        val = jnp.minimum(val[:s], val[s:])
        s //= 2
    m8 = jnp.min(val, axis=0, keepdims=True)
    rows = lax.broadcasted_iota(jnp.int32, val.shape, 0)
    idx = jnp.min(jnp.where(val == m8, rows + acc, _K), axis=0)
    idx_ref[...] = idx[None, :]

    ids = lax.broadcasted_iota(jnp.int32, (_K, _TB), 0)

    one_hot = (ids == idx[None, :]).astype(jnp.bfloat16)
    q = lax.dot_general(wb_ref[...], one_hot,
                        (((1,), (0,)), ((), ())),
                        preferred_element_type=jnp.float32)
    qt_ref[...] = xt + (q - xt)

    part = jnp.sum((q - xt) * (q - xt)).reshape(1, 1)

    @pl.when(i == 0)
    def _():
        loss_ref[...] = jnp.zeros((1, 1), jnp.float32)

    loss_ref[...] += part

    @pl.when(i == _GRID - 1)
    def _():
        s = loss_ref[...] / jnp.float32(_B * _D)
        loss_ref[...] = s * _BETA + s


@jax.jit
def kernel(x, W):
    idxt, qt, loss = pl.pallas_call(
        _vq_body,
        grid=(_GRID,),
        in_specs=[
            pl.BlockSpec((_D, _TB), lambda i: (0, i)),
            pl.BlockSpec((_D, _K), lambda i: (0, 0)),
        ],
        out_specs=[
            pl.BlockSpec((1, _TB), lambda i: (0, i)),
            pl.BlockSpec((_D, _TB), lambda i: (0, i)),
            pl.BlockSpec((1, 1), lambda i: (0, 0)),
        ],
        out_shape=[
            jax.ShapeDtypeStruct((1, _B), jnp.int32),
            jax.ShapeDtypeStruct((_D, _B), jnp.float32),
            jax.ShapeDtypeStruct((1, 1), jnp.float32),
        ],
        scratch_shapes=[
            pltpu.VMEM((_K, 1), jnp.float32),
            pltpu.VMEM((_D, _K), jnp.bfloat16),
        ],
    )(x.T, W.T)
    return idxt.T, qt.T, loss[0, 0]
